# baseline (speedup 1.0000x reference)
import functools

import jax
import jax.numpy as jnp
from jax.experimental import pallas as pl
from jax.experimental.pallas import tpu as pltpu

B = 16
D = 32
HID = 128
POS = 512 * 512
ROWS = POS * D // 128
BR = 8192
NSTEPS = ROWS // BR
ACC = 64


def _reduce_body(x_ref, acc_ref):
    j = pl.program_id(1)

    @pl.when(j == 0)
    def _():
        acc_ref[...] = jnp.zeros_like(acc_ref)

    x = x_ref[0]
    acc_ref[0] += jnp.sum(x.reshape(BR // ACC, ACC, 128), axis=0)


def _head_body(p_ref, w1_ref, b1_ref, w2_ref, b2_ref, g_ref,
               rout_ref, probs_ref, idx_ref):
    p = jnp.sum(p_ref[...], axis=1)
    x = (p[:, 0:32] + p[:, 32:64] + p[:, 64:96] + p[:, 96:128]) * (1.0 / POS)
    h = jnp.maximum(
        jax.lax.dot_general(x, w1_ref[...], (((1,), (0,)), ((), ())),
                            preferred_element_type=jnp.float32) + b1_ref[...],
        0.0)
    logits = jax.lax.dot_general(h, w2_ref[...], (((1,), (0,)), ((), ())),
                                 preferred_element_type=jnp.float32) + b2_ref[...]
    m = jnp.max(logits, axis=-1, keepdims=True)
    e = jnp.exp(logits - m)
    probs = e / jnp.sum(e, axis=-1, keepdims=True)
    probs_ref[...] = probs
    z = jnp.log(probs + 1e-20) + g_ref[...]
    zmax = jnp.max(z, axis=-1, keepdims=True)
    lane = jax.lax.broadcasted_iota(jnp.int32, (B, D), 1)
    idx = jnp.min(jnp.where(z >= zmax, lane, D), axis=-1, keepdims=True)
    idx_ref[...] = idx
    rout_ref[...] = (lane == idx).astype(jnp.float32)


GRP = 8
PER = B // GRP


@functools.partial(jax.jit, static_argnames=())
def kernel(attn_5d, W1, b1, W2, b2):
    parts = []
    for g in range(GRP):
        xg = jax.lax.slice_in_dim(attn_5d, g * PER, (g + 1) * PER, axis=0)
        xg = xg.reshape(PER, ROWS, 128)
        parts.append(pl.pallas_call(
            _reduce_body,
            grid=(PER, NSTEPS),
            in_specs=[pl.BlockSpec((1, BR, 128), lambda b, j: (b, j, 0))],
            out_specs=pl.BlockSpec((1, ACC, 128), lambda b, j: (b, 0, 0)),
            out_shape=jax.ShapeDtypeStruct((PER, ACC, 128), jnp.float32),
        )(xg))
    partial = jnp.concatenate(parts, axis=0)

    gumbel = jax.random.gumbel(jax.random.key(1234), (B, D), jnp.float32)
    rout, probs, idx = pl.pallas_call(
        _head_body,
        in_specs=[
            pl.BlockSpec((B, ACC, 128), lambda: (0, 0, 0)),
            pl.BlockSpec((D, HID), lambda: (0, 0)),
            pl.BlockSpec((1, HID), lambda: (0, 0)),
            pl.BlockSpec((HID, D), lambda: (0, 0)),
            pl.BlockSpec((1, D), lambda: (0, 0)),
            pl.BlockSpec((B, D), lambda: (0, 0)),
        ],
        out_specs=[
            pl.BlockSpec((B, D), lambda: (0, 0)),
            pl.BlockSpec((B, D), lambda: (0, 0)),
            pl.BlockSpec((B, 1), lambda: (0, 0)),
        ],
        out_shape=[
            jax.ShapeDtypeStruct((B, D), jnp.float32),
            jax.ShapeDtypeStruct((B, D), jnp.float32),
            jax.ShapeDtypeStruct((B, 1), jnp.int32),
        ],
    )(partial, W1, b1.reshape(1, HID), W2, b2.reshape(1, D), gumbel)
    return rout, probs, idx.reshape(B)


# speedup vs baseline: 1.5872x; 1.5872x over previous
import functools

import jax
import jax.numpy as jnp
from jax.experimental import pallas as pl
from jax.experimental.pallas import tpu as pltpu

B = 16
D = 32
HID = 128
POS = 512 * 512
ROWS = POS * D // 128
BR = 8192
NSTEPS = ROWS // BR
ACC = 64


def _reduce_body(x_ref, acc_ref):
    j = pl.program_id(1)

    @pl.when(j == 0)
    def _():
        acc_ref[...] = jnp.zeros_like(acc_ref)

    x = x_ref[0]
    acc_ref[0] += jnp.sum(x.reshape(BR // ACC, ACC, 128), axis=0)


def _head_body(p_ref, w1_ref, b1_ref, w2_ref, b2_ref, g_ref,
               rout_ref, probs_ref, idx_ref):
    p = jnp.sum(p_ref[...], axis=1)
    x = (p[:, 0:32] + p[:, 32:64] + p[:, 64:96] + p[:, 96:128]) * (1.0 / POS)
    h = jnp.maximum(
        jax.lax.dot_general(x, w1_ref[...], (((1,), (0,)), ((), ())),
                            preferred_element_type=jnp.float32) + b1_ref[...],
        0.0)
    logits = jax.lax.dot_general(h, w2_ref[...], (((1,), (0,)), ((), ())),
                                 preferred_element_type=jnp.float32) + b2_ref[...]
    m = jnp.max(logits, axis=-1, keepdims=True)
    e = jnp.exp(logits - m)
    probs = e / jnp.sum(e, axis=-1, keepdims=True)
    probs_ref[...] = probs
    z = jnp.log(probs + 1e-20) + g_ref[...]
    zmax = jnp.max(z, axis=-1, keepdims=True)
    lane = jax.lax.broadcasted_iota(jnp.int32, (B, D), 1)
    idx = jnp.min(jnp.where(z >= zmax, lane, D), axis=-1, keepdims=True)
    idx_ref[...] = idx
    rout_ref[...] = (lane == idx).astype(jnp.float32)


@functools.partial(jax.jit, static_argnames=())
def kernel(attn_5d, W1, b1, W2, b2):
    x = attn_5d.reshape(B, ROWS, 128)
    partial = pl.pallas_call(
        _reduce_body,
        grid=(B, NSTEPS),
        in_specs=[pl.BlockSpec((1, BR, 128), lambda b, j: (b, j, 0))],
        out_specs=pl.BlockSpec((1, ACC, 128), lambda b, j: (b, 0, 0)),
        out_shape=jax.ShapeDtypeStruct((B, ACC, 128), jnp.float32),
    )(x)

    gumbel = jax.random.gumbel(jax.random.key(1234), (B, D), jnp.float32)
    rout, probs, idx = pl.pallas_call(
        _head_body,
        in_specs=[
            pl.BlockSpec((B, ACC, 128), lambda: (0, 0, 0)),
            pl.BlockSpec((D, HID), lambda: (0, 0)),
            pl.BlockSpec((1, HID), lambda: (0, 0)),
            pl.BlockSpec((HID, D), lambda: (0, 0)),
            pl.BlockSpec((1, D), lambda: (0, 0)),
            pl.BlockSpec((B, D), lambda: (0, 0)),
        ],
        out_specs=[
            pl.BlockSpec((B, D), lambda: (0, 0)),
            pl.BlockSpec((B, D), lambda: (0, 0)),
            pl.BlockSpec((B, 1), lambda: (0, 0)),
        ],
        out_shape=[
            jax.ShapeDtypeStruct((B, D), jnp.float32),
            jax.ShapeDtypeStruct((B, D), jnp.float32),
            jax.ShapeDtypeStruct((B, 1), jnp.int32),
        ],
    )(partial, W1, b1.reshape(1, HID), W2, b2.reshape(1, D), gumbel)
    return rout, probs, idx.reshape(B)


# speedup vs baseline: 2.9938x; 1.8862x over previous
import functools

import jax
import jax.numpy as jnp
from jax.experimental import pallas as pl
from jax.experimental.pallas import tpu as pltpu

B = 16
D = 32
HID = 128
HH = 512
POS = 512 * 512
ROWLEN = 512 * D
BH = 64
NSTEPS = HH // BH
ACC = 64


def _reduce_body(x_ref, acc_ref):
    j = pl.program_id(1)

    @pl.when(j == 0)
    def _():
        acc_ref[...] = jnp.zeros_like(acc_ref)

    x = x_ref[0]
    acc_ref[0] += jnp.sum(x.reshape(BH * ROWLEN // (ACC * 128), ACC, 128), axis=0)


def _head_body(p_ref, w1_ref, b1_ref, w2_ref, b2_ref, g_ref,
               rout_ref, probs_ref, idx_ref):
    p = jnp.sum(p_ref[...], axis=1)
    x = (p[:, 0:32] + p[:, 32:64] + p[:, 64:96] + p[:, 96:128]) * (1.0 / POS)
    h = jnp.maximum(
        jax.lax.dot_general(x, w1_ref[...], (((1,), (0,)), ((), ())),
                            preferred_element_type=jnp.float32) + b1_ref[...],
        0.0)
    logits = jax.lax.dot_general(h, w2_ref[...], (((1,), (0,)), ((), ())),
                                 preferred_element_type=jnp.float32) + b2_ref[...]
    m = jnp.max(logits, axis=-1, keepdims=True)
    e = jnp.exp(logits - m)
    probs = e / jnp.sum(e, axis=-1, keepdims=True)
    probs_ref[...] = probs
    z = jnp.log(probs + 1e-20) + g_ref[...]
    zmax = jnp.max(z, axis=-1, keepdims=True)
    lane = jax.lax.broadcasted_iota(jnp.int32, (B, D), 1)
    idx = jnp.min(jnp.where(z >= zmax, lane, D), axis=-1, keepdims=True)
    idx_ref[...] = idx
    rout_ref[...] = (lane == idx).astype(jnp.float32)


@functools.partial(jax.jit, static_argnames=())
def kernel(attn_5d, W1, b1, W2, b2):
    x = attn_5d.reshape(B, HH, ROWLEN)
    partial = pl.pallas_call(
        _reduce_body,
        grid=(B, NSTEPS),
        in_specs=[pl.BlockSpec((1, BH, ROWLEN), lambda b, j: (b, j, 0))],
        out_specs=pl.BlockSpec((1, ACC, 128), lambda b, j: (b, 0, 0)),
        out_shape=jax.ShapeDtypeStruct((B, ACC, 128), jnp.float32),
    )(x)

    gumbel = jax.random.gumbel(jax.random.key(1234), (B, D), jnp.float32)
    rout, probs, idx = pl.pallas_call(
        _head_body,
        in_specs=[
            pl.BlockSpec((B, ACC, 128), lambda: (0, 0, 0)),
            pl.BlockSpec((D, HID), lambda: (0, 0)),
            pl.BlockSpec((1, HID), lambda: (0, 0)),
            pl.BlockSpec((HID, D), lambda: (0, 0)),
            pl.BlockSpec((1, D), lambda: (0, 0)),
            pl.BlockSpec((B, D), lambda: (0, 0)),
        ],
        out_specs=[
            pl.BlockSpec((B, D), lambda: (0, 0)),
            pl.BlockSpec((B, D), lambda: (0, 0)),
            pl.BlockSpec((B, 1), lambda: (0, 0)),
        ],
        out_shape=[
            jax.ShapeDtypeStruct((B, D), jnp.float32),
            jax.ShapeDtypeStruct((B, D), jnp.float32),
            jax.ShapeDtypeStruct((B, 1), jnp.int32),
        ],
    )(partial, W1, b1.reshape(1, HID), W2, b2.reshape(1, D), gumbel)
    return rout, probs, idx.reshape(B)
